# Initial kernel scaffold; baseline (speedup 1.0000x reference)
import jax
import jax.numpy as jnp
from jax.experimental import pallas as pl


def kernel(x_num, x_cat, tables, W1, b1, g1, be1, W2, b2, g2, be2, W3, b3):
    raise NotImplementedError("write your pallas kernel here")


# speedup vs baseline: 7.1077x; 7.1077x over previous
import functools

import jax
import jax.numpy as jnp
from jax import lax
from jax.experimental import pallas as pl
from jax.experimental.pallas import tpu as pltpu
from jax.experimental.pallas import tpu_sc as plsc

B = 16384
NUM_NUMERIC = 16
N_CAT = 26
VOCAB = 100000
EMB = 16
EMB_TOTAL = N_CAT * EMB
IN_DIM = NUM_NUMERIC + EMB_TOTAL
N_FLAT = B * N_CAT

NC = 2
NS = 16
NW = NC * NS
PER_W = N_FLAT // NW
CHUNK = 128
N_CHUNKS = PER_W // CHUNK


def _gather_body(tab_ref, idx_ref, out_ref, idx_v, fidx_v, rows_v, sem):
    wid = lax.axis_index("s") * NC + lax.axis_index("c")
    base = wid * PER_W

    def chunk(j, carry):
        start = base + j * CHUNK
        pltpu.sync_copy(idx_ref.at[pl.ds(start, CHUNK)], idx_v)
        for t in range(CHUNK // 16):
            pos = start + t * 16 + lax.iota(jnp.int32, 16)
            col = lax.rem(pos, N_CAT)
            fidx_v[pl.ds(t * 16, 16)] = idx_v[pl.ds(t * 16, 16)] + col * VOCAB
        pltpu.async_copy(tab_ref.at[fidx_v], rows_v, sem).wait()
        pltpu.sync_copy(rows_v, out_ref.at[pl.ds(start, CHUNK)])
        return carry

    lax.fori_loop(0, N_CHUNKS, chunk, 0)


_sc_gather = pl.kernel(
    _gather_body,
    mesh=plsc.VectorSubcoreMesh(core_axis_name="c", subcore_axis_name="s"),
    compiler_params=pltpu.CompilerParams(use_tc_tiling_on_sc=False),
    out_type=jax.ShapeDtypeStruct((N_FLAT, EMB), jnp.float32),
    scratch_types=[
        pltpu.VMEM((CHUNK,), jnp.int32),
        pltpu.VMEM((CHUNK,), jnp.int32),
        pltpu.VMEM((CHUNK, EMB), jnp.float32),
        pltpu.SemaphoreType.DMA,
    ],
)


BLK = 2048


def _mlp_body(xn_ref, xe_ref, w1n_ref, w1e_ref, b1_ref, g1_ref, be1_ref,
              w2_ref, b2_ref, g2_ref, be2_ref, w3_ref, b3_ref, out_ref):
    f32 = jnp.float32
    h = jnp.dot(xn_ref[...], w1n_ref[...], preferred_element_type=f32)
    h += jnp.dot(xe_ref[...], w1e_ref[...], preferred_element_type=f32)
    h = jnp.maximum(h + b1_ref[...], 0.0)
    mu = jnp.mean(h, axis=1, keepdims=True)
    d = h - mu
    var = jnp.mean(d * d, axis=1, keepdims=True)
    h = d * lax.rsqrt(var + 1e-5) * g1_ref[...] + be1_ref[...]
    h = jnp.maximum(jnp.dot(h, w2_ref[...], preferred_element_type=f32)
                    + b2_ref[...], 0.0)
    mu = jnp.mean(h, axis=1, keepdims=True)
    d = h - mu
    var = jnp.mean(d * d, axis=1, keepdims=True)
    h = d * lax.rsqrt(var + 1e-5) * g2_ref[...] + be2_ref[...]
    out_ref[...] = jnp.sum(h * w3_ref[...], axis=1) + b3_ref[0, 0]


def _mlp(x_num, embs, W1n, W1e, b1, g1, be1, W2, b2, g2, be2, W3r, b3):
    full = lambda r, c: pl.BlockSpec((r, c), lambda i: (0, 0))
    return pl.pallas_call(
        _mlp_body,
        grid=(B // BLK,),
        in_specs=[
            pl.BlockSpec((BLK, NUM_NUMERIC), lambda i: (i, 0)),
            pl.BlockSpec((BLK, EMB_TOTAL), lambda i: (i, 0)),
            full(NUM_NUMERIC, 64), full(EMB_TOTAL, 64),
            full(1, 64), full(1, 64), full(1, 64),
            full(64, 32), full(1, 32), full(1, 32), full(1, 32),
            full(1, 32), full(1, 1),
        ],
        out_specs=pl.BlockSpec((BLK,), lambda i: (i,)),
        out_shape=jax.ShapeDtypeStruct((B,), jnp.float32),
    )(x_num, embs, W1n, W1e, b1, g1, be1, W2, b2, g2, be2, W3r, b3)


def kernel(x_num, x_cat, tables, W1, b1, g1, be1, W2, b2, g2, be2, W3, b3):
    tab_flat = tables.reshape(N_CAT * VOCAB, EMB)
    idx_flat = x_cat.reshape(N_FLAT).astype(jnp.int32)
    embs = _sc_gather(tab_flat, idx_flat).reshape(B, EMB_TOTAL)
    r1 = lambda v: v.reshape(1, -1)
    return _mlp(x_num, embs,
                W1[:NUM_NUMERIC], W1[NUM_NUMERIC:], r1(b1), r1(g1), r1(be1),
                W2, r1(b2), r1(g2), r1(be2), r1(W3), r1(b3))
